# Initial kernel scaffold; baseline (speedup 1.0000x reference)
import jax
import jax.numpy as jnp
from jax.experimental import pallas as pl


def kernel(x1, edge_index1, batch1, x2, edge_index2, batch2, W1, b1, W2, b2, Wc1, bc1, Wc2, bc2):
    raise NotImplementedError("write your pallas kernel here")


# speedup vs baseline: 8.5308x; 8.5308x over previous
import functools

import jax
import jax.numpy as jnp
from jax import lax
from jax.experimental import pallas as pl
from jax.experimental.pallas import tpu as pltpu
from jax.experimental.pallas import tpu_sc as plsc

N = 10000
E = 320000
B = 512
IN_DIM = 128
HID = 256

NC = 2
NS = 16
CH = 80
EPT = E // NS
NCH = EPT // CH
RPT0 = 640
RPT_LAST = N - 15 * RPT0

_mesh = plsc.VectorSubcoreMesh(core_axis_name="c", subcore_axis_name="s",
                               num_cores=NC, num_subcores=NS)


@functools.partial(
    pl.kernel,
    out_type=jax.ShapeDtypeStruct((NC * N, 128), jnp.float32),
    mesh=_mesh,
    scratch_types=[
        pltpu.VMEM((CH,), jnp.int32),
        pltpu.VMEM((CH, 128), jnp.float32),
        pltpu.VMEM((CH, 128), jnp.float32),
        pltpu.VMEM_SHARED((N, 128), jnp.float32),
    ],
)
def _deg_kernel(dst_hbm, deg_out, idx_v, ones_v, zer_v, acc):
    c = lax.axis_index("c")
    s = lax.axis_index("s")

    def fill_ones(i, _):
        for k in range(8):
            ones_v[i, pl.ds(k * 16, 16)] = jnp.ones((16,), jnp.float32)
            zer_v[i, pl.ds(k * 16, 16)] = jnp.zeros((16,), jnp.float32)
        return 0

    lax.fori_loop(0, CH, fill_ones, 0)

    @pl.when(s < 15)
    def _():
        for k in range(RPT0 // CH):
            pltpu.sync_copy(zer_v, acc.at[pl.ds(s * RPT0 + k * CH, CH)])

    @pl.when(s == 15)
    def _():
        for k in range(RPT_LAST // CH):
            pltpu.sync_copy(zer_v, acc.at[pl.ds(15 * RPT0 + k * CH, CH)])

    plsc.subcore_barrier()

    def body(j, _):
        base = c * E + s * EPT + j * CH
        pltpu.sync_copy(dst_hbm.at[pl.ds(base, CH)], idx_v)
        pltpu.sync_copy(ones_v, acc.at[idx_v], add=True)
        return 0

    lax.fori_loop(0, NCH, body, 0)
    plsc.subcore_barrier()

    @pl.when(s < 15)
    def _():
        pltpu.sync_copy(acc.at[pl.ds(s * RPT0, RPT0)],
                        deg_out.at[pl.ds(c * N + s * RPT0, RPT0)])

    @pl.when(s == 15)
    def _():
        pltpu.sync_copy(acc.at[pl.ds(15 * RPT0, RPT_LAST)],
                        deg_out.at[pl.ds(c * N + 15 * RPT0, RPT_LAST)])


def _make_prop(n_jobs):
    n_rounds = n_jobs // NC

    @functools.partial(
        pl.kernel,
        out_type=jax.ShapeDtypeStruct((n_jobs * N, 128), jnp.float32),
        mesh=_mesh,
        scratch_types=[
            pltpu.VMEM((CH,), jnp.int32),
            pltpu.VMEM((CH,), jnp.int32),
            pltpu.VMEM((CH, 128), jnp.float32),
            pltpu.VMEM((CH, 128), jnp.float32),
            pltpu.VMEM_SHARED((N, 128), jnp.float32),
            pltpu.SemaphoreType.DMA,
        ],
    )
    def prop(table_hbm, src_hbm, dst_hbm, out_hbm,
             src_v, dst_v, rows_v, zer_v, acc, sem):
        c = lax.axis_index("c")
        s = lax.axis_index("s")

        def fill_zero(i, _):
            for k in range(8):
                zer_v[i, pl.ds(k * 16, 16)] = jnp.zeros((16,), jnp.float32)
            return 0

        lax.fori_loop(0, CH, fill_zero, 0)

        def zero_acc():
            @pl.when(s < 15)
            def _():
                for k in range(RPT0 // CH):
                    pltpu.sync_copy(zer_v, acc.at[pl.ds(s * RPT0 + k * CH, CH)])

            @pl.when(s == 15)
            def _():
                for k in range(RPT_LAST // CH):
                    pltpu.sync_copy(zer_v, acc.at[pl.ds(15 * RPT0 + k * CH, CH)])

        zero_acc()
        plsc.subcore_barrier()

        for r in range(n_rounds):
            job = r * NC + c

            def body(j, _):
                base = job * E + s * EPT + j * CH
                pltpu.sync_copy(src_hbm.at[pl.ds(base, CH)], src_v)
                pltpu.sync_copy(dst_hbm.at[pl.ds(base, CH)], dst_v)
                pltpu.async_copy(table_hbm.at[src_v], rows_v, sem).wait()
                pltpu.sync_copy(rows_v, acc.at[dst_v], add=True)
                return 0

            lax.fori_loop(0, NCH, body, 0)
            plsc.subcore_barrier()

            @pl.when(s < 15)
            def _():
                pltpu.sync_copy(acc.at[pl.ds(s * RPT0, RPT0)],
                                out_hbm.at[pl.ds(job * N + s * RPT0, RPT0)])

            @pl.when(s == 15)
            def _():
                pltpu.sync_copy(acc.at[pl.ds(15 * RPT0, RPT_LAST)],
                                out_hbm.at[pl.ds(job * N + 15 * RPT0, RPT_LAST)])

            if r + 1 < n_rounds:
                zero_acc()
                plsc.subcore_barrier()

    return prop


_prop2 = _make_prop(2)
_prop4 = _make_prop(4)


_BN = 2000
_NB = N // _BN


def _prescale_body(x_ref, deg_ref, o_ref):
    d = deg_ref[0][:, 0:1]
    dinv = lax.rsqrt(d + 1.0)
    o_ref[0] = x_ref[0] * dinv


_prescale = pl.pallas_call(
    _prescale_body,
    grid=(2, _NB),
    in_specs=[pl.BlockSpec((1, _BN, IN_DIM), lambda g, i: (g, i, 0)),
              pl.BlockSpec((1, _BN, 128), lambda g, i: (g, i, 0))],
    out_specs=pl.BlockSpec((1, _BN, IN_DIM), lambda g, i: (g, i, 0)),
    out_shape=jax.ShapeDtypeStruct((2, N, IN_DIM), jnp.float32),
)


def _conv1_body(tmp_ref, xs_ref, deg_ref, w_ref, b_ref, o_ref):
    d = deg_ref[0][:, 0:1]
    dinv = lax.rsqrt(d + 1.0)
    p = dinv * (tmp_ref[0] + xs_ref[0])
    h = jnp.dot(p, w_ref[...], preferred_element_type=jnp.float32,
                 precision=lax.Precision.HIGHEST)
    h = jnp.maximum(h + b_ref[0], 0.0)
    hs = h * dinv
    o_ref[0, 0] = hs[:, :128]
    o_ref[0, 1] = hs[:, 128:]


_conv1 = pl.pallas_call(
    _conv1_body,
    grid=(2, _NB),
    in_specs=[pl.BlockSpec((1, _BN, IN_DIM), lambda g, i: (g, i, 0)),
              pl.BlockSpec((1, _BN, IN_DIM), lambda g, i: (g, i, 0)),
              pl.BlockSpec((1, _BN, 128), lambda g, i: (g, i, 0)),
              pl.BlockSpec((IN_DIM, HID), lambda g, i: (0, 0)),
              pl.BlockSpec((1, HID), lambda g, i: (0, 0))],
    out_specs=pl.BlockSpec((1, 2, _BN, 128), lambda g, i: (g, 0, i, 0)),
    out_shape=jax.ShapeDtypeStruct((2, 2, N, 128), jnp.float32),
)


def _conv2_pool_body(tmp_ref, hs_ref, deg_ref, w_ref, b_ref, batch_ref, o_ref):
    i = pl.program_id(1)
    d = deg_ref[0][:, 0:1]
    dinv = lax.rsqrt(d + 1.0)
    p_lo = dinv * (tmp_ref[0, 0] + hs_ref[0, 0])
    p_hi = dinv * (tmp_ref[0, 1] + hs_ref[0, 1])
    h2 = jnp.dot(p_lo, w_ref[:128, :], preferred_element_type=jnp.float32,
                 precision=lax.Precision.HIGHEST)
    h2 = h2 + jnp.dot(p_hi, w_ref[128:, :], preferred_element_type=jnp.float32,
                 precision=lax.Precision.HIGHEST)
    h2 = jnp.maximum(h2 + b_ref[0], 0.0)
    bvec = batch_ref[0, 0, 0, :]
    seg = lax.broadcasted_iota(jnp.int32, (B, _BN), 0)
    m = (seg == bvec[None, :]).astype(jnp.float32)
    h2e = jnp.concatenate([h2, jnp.ones((_BN, 128), jnp.float32)], axis=1)
    part = jnp.dot(m, h2e, preferred_element_type=jnp.float32,
                 precision=lax.Precision.HIGHEST)

    @pl.when(i == 0)
    def _():
        o_ref[0] = part

    @pl.when(i > 0)
    def _():
        o_ref[0] += part


_conv2_pool = pl.pallas_call(
    _conv2_pool_body,
    grid=(2, _NB),
    in_specs=[pl.BlockSpec((1, 2, _BN, 128), lambda g, i: (g, 0, i, 0)),
              pl.BlockSpec((1, 2, _BN, 128), lambda g, i: (g, 0, i, 0)),
              pl.BlockSpec((1, _BN, 128), lambda g, i: (g, i, 0)),
              pl.BlockSpec((HID, HID), lambda g, i: (0, 0)),
              pl.BlockSpec((1, HID), lambda g, i: (0, 0)),
              pl.BlockSpec((1, 1, 1, _BN), lambda g, i: (g, i, 0, 0))],
    out_specs=pl.BlockSpec((1, B, HID + 128), lambda g, i: (g, 0, 0)),
    out_shape=jax.ShapeDtypeStruct((2, B, HID + 128), jnp.float32),
)


def _cls_body(s_ref, wc1_ref, bc1_ref, wc2_ref, bc2_ref, o_ref):
    s1 = s_ref[0]
    s2 = s_ref[1]
    g1 = s1[:, :HID] / jnp.maximum(s1[:, HID:HID + 1], 1.0)
    g2 = s2[:, :HID] / jnp.maximum(s2[:, HID:HID + 1], 1.0)
    pair = jnp.concatenate([g1 * g2, jnp.abs(g1 - g2)], axis=1)
    hid = jnp.dot(pair, wc1_ref[...], preferred_element_type=jnp.float32,
                 precision=lax.Precision.HIGHEST)
    hid = jnp.maximum(hid + bc1_ref[0], 0.0)
    r = jnp.dot(hid, wc2_ref[...], preferred_element_type=jnp.float32,
                 precision=lax.Precision.HIGHEST)
    r = r + bc2_ref[0]
    o_ref[...] = jnp.broadcast_to(r, (B, 128))


_cls = pl.pallas_call(
    _cls_body,
    out_shape=jax.ShapeDtypeStruct((B, 128), jnp.float32),
)


def kernel(x1, edge_index1, batch1, x2, edge_index2, batch2,
           W1, b1, W2, b2, Wc1, bc1, Wc2, bc2):
    src1, dst1 = edge_index1[0], edge_index1[1]
    src2, dst2 = edge_index2[0], edge_index2[1]

    dst_cat = jnp.concatenate([dst1, dst2])
    src_cat = jnp.concatenate([src1, src2 + N])
    src4 = jnp.concatenate([src1, src1 + N, src2 + 2 * N, src2 + 3 * N])
    dst4 = jnp.concatenate([dst1, dst1, dst2, dst2])

    deg = _deg_kernel(dst_cat).reshape(2, N, 128)
    X = jnp.stack([x1, x2])
    XS = _prescale(X, deg)
    tmp1 = _prop2(XS.reshape(2 * N, 128), src_cat, dst_cat)
    HS4 = _conv1(tmp1.reshape(2, N, 128), XS, deg, W1, b1.reshape(1, HID))
    tmp2 = _prop4(HS4.reshape(4 * N, 128), src4, dst4)
    batch4 = jnp.stack([batch1, batch2]).reshape(2, _NB, 1, _BN)
    sums = _conv2_pool(tmp2.reshape(2, 2, N, 128), HS4, deg, W2,
                       b2.reshape(1, HID), batch4)
    out = _cls(sums, Wc1, bc1.reshape(1, HID), Wc2, bc2.reshape(1, 1))
    return out[:, 0]


# speedup vs baseline: 19.0139x; 2.2289x over previous
import functools

import jax
import jax.numpy as jnp
from jax import lax
from jax.experimental import pallas as pl
from jax.experimental.pallas import tpu as pltpu
from jax.experimental.pallas import tpu_sc as plsc

N = 10000
E = 320000
B = 512
IN_DIM = 128
HID = 256

NC = 2
NS = 16
CH = 80
EPT = E // NS
NCH = EPT // CH
RPT0 = 640
RPT_LAST = N - 15 * RPT0
NBLK = 5
IBE = EPT // NBLK
BCH = IBE // CH

_mesh = plsc.VectorSubcoreMesh(core_axis_name="c", subcore_axis_name="s",
                               num_cores=NC, num_subcores=NS)


@functools.partial(
    pl.kernel,
    out_type=jax.ShapeDtypeStruct((NC * N, 128), jnp.float32),
    mesh=_mesh,
    scratch_types=[
        pltpu.VMEM((EPT,), jnp.int32),
        pltpu.VMEM((CH, 128), jnp.float32),
        pltpu.VMEM((CH, 128), jnp.float32),
        pltpu.VMEM_SHARED((N, 128), jnp.float32),
    ],
)
def _deg_kernel(dst_hbm, deg_out, idx_v, ones_v, zer_v, acc):
    c = lax.axis_index("c")
    s = lax.axis_index("s")

    def fill_ones(i, _):
        for k in range(8):
            ones_v[i, pl.ds(k * 16, 16)] = jnp.ones((16,), jnp.float32)
            zer_v[i, pl.ds(k * 16, 16)] = jnp.zeros((16,), jnp.float32)
        return 0

    lax.fori_loop(0, CH, fill_ones, 0)

    @pl.when(s < 15)
    def _():
        for k in range(RPT0 // CH):
            pltpu.sync_copy(zer_v, acc.at[pl.ds(s * RPT0 + k * CH, CH)])

    @pl.when(s == 15)
    def _():
        for k in range(RPT_LAST // CH):
            pltpu.sync_copy(zer_v, acc.at[pl.ds(15 * RPT0 + k * CH, CH)])

    plsc.subcore_barrier()

    pltpu.sync_copy(dst_hbm.at[pl.ds(c * E + s * EPT, EPT)], idx_v)

    def body(j, _):
        pltpu.sync_copy(ones_v, acc.at[idx_v.at[pl.ds(j * CH, CH)]],
                        add=True)
        return 0

    lax.fori_loop(0, NCH, body, 0)
    plsc.subcore_barrier()

    @pl.when(s < 15)
    def _():
        pltpu.sync_copy(acc.at[pl.ds(s * RPT0, RPT0)],
                        deg_out.at[pl.ds(c * N + s * RPT0, RPT0)])

    @pl.when(s == 15)
    def _():
        pltpu.sync_copy(acc.at[pl.ds(15 * RPT0, RPT_LAST)],
                        deg_out.at[pl.ds(c * N + 15 * RPT0, RPT_LAST)])


def _make_prop(n_jobs):
    n_rounds = n_jobs // NC

    @functools.partial(
        pl.kernel,
        out_type=jax.ShapeDtypeStruct((n_jobs * N, 128), jnp.float32),
        mesh=_mesh,
        scratch_types=[
            pltpu.VMEM((IBE,), jnp.int32),
            pltpu.VMEM((IBE,), jnp.int32),
            pltpu.VMEM((CH, 128), jnp.float32),
            pltpu.VMEM((CH, 128), jnp.float32),
            pltpu.VMEM((CH, 128), jnp.float32),
            pltpu.VMEM_SHARED((N, 128), jnp.float32),
            pltpu.SemaphoreType.DMA,
            pltpu.SemaphoreType.DMA,
        ],
    )
    def prop(table_hbm, src_hbm, dst_hbm, out_hbm,
             src_v, dst_v, rows_a, rows_b, zer_v, acc, sem_a, sem_b):
        c = lax.axis_index("c")
        s = lax.axis_index("s")

        def fill_zero(i, _):
            for k in range(8):
                zer_v[i, pl.ds(k * 16, 16)] = jnp.zeros((16,), jnp.float32)
            return 0

        lax.fori_loop(0, CH, fill_zero, 0)

        def zero_acc():
            @pl.when(s < 15)
            def _():
                for k in range(RPT0 // CH):
                    pltpu.sync_copy(zer_v, acc.at[pl.ds(s * RPT0 + k * CH, CH)])

            @pl.when(s == 15)
            def _():
                for k in range(RPT_LAST // CH):
                    pltpu.sync_copy(zer_v, acc.at[pl.ds(15 * RPT0 + k * CH, CH)])

        zero_acc()
        plsc.subcore_barrier()

        for r in range(n_rounds):
            job = r * NC + c
            base = job * E + s * EPT

            def gather(j, buf, sem):
                return pltpu.async_copy(
                    table_hbm.at[src_v.at[pl.ds(j * CH, CH)]], buf, sem)

            def scat(j, buf):
                pltpu.sync_copy(buf, acc.at[dst_v.at[pl.ds(j * CH, CH)]],
                                add=True)

            def blk(bi, _):
                bbase = base + bi * IBE
                pltpu.sync_copy(src_hbm.at[pl.ds(bbase, IBE)], src_v)
                pltpu.sync_copy(dst_hbm.at[pl.ds(bbase, IBE)], dst_v)
                gather(0, rows_a, sem_a)

                def body(i, _):
                    ja = 2 * i
                    gather(ja + 1, rows_b, sem_b)
                    pltpu.make_async_copy(
                        table_hbm.at[src_v.at[pl.ds(ja * CH, CH)]],
                        rows_a, sem_a).wait()
                    scat(ja, rows_a)

                    @pl.when(i + 1 < BCH // 2)
                    def _():
                        gather(ja + 2, rows_a, sem_a)

                    pltpu.make_async_copy(
                        table_hbm.at[src_v.at[pl.ds((ja + 1) * CH, CH)]],
                        rows_b, sem_b).wait()
                    scat(ja + 1, rows_b)
                    return 0

                lax.fori_loop(0, BCH // 2, body, 0)
                return 0

            lax.fori_loop(0, NBLK, blk, 0)
            plsc.subcore_barrier()

            @pl.when(s < 15)
            def _():
                pltpu.sync_copy(acc.at[pl.ds(s * RPT0, RPT0)],
                                out_hbm.at[pl.ds(job * N + s * RPT0, RPT0)])

            @pl.when(s == 15)
            def _():
                pltpu.sync_copy(acc.at[pl.ds(15 * RPT0, RPT_LAST)],
                                out_hbm.at[pl.ds(job * N + 15 * RPT0, RPT_LAST)])

            if r + 1 < n_rounds:
                zero_acc()
                plsc.subcore_barrier()

    return prop


_prop2 = _make_prop(2)
_prop4 = _make_prop(4)


_BN = 2000
_NB = N // _BN


def _prescale_body(x_ref, deg_ref, o_ref):
    d = deg_ref[0][:, 0:1]
    dinv = lax.rsqrt(d + 1.0)
    o_ref[0] = x_ref[0] * dinv


_prescale = pl.pallas_call(
    _prescale_body,
    grid=(2, _NB),
    in_specs=[pl.BlockSpec((1, _BN, IN_DIM), lambda g, i: (g, i, 0)),
              pl.BlockSpec((1, _BN, 128), lambda g, i: (g, i, 0))],
    out_specs=pl.BlockSpec((1, _BN, IN_DIM), lambda g, i: (g, i, 0)),
    out_shape=jax.ShapeDtypeStruct((2, N, IN_DIM), jnp.float32),
)


def _conv1_body(tmp_ref, xs_ref, deg_ref, w_ref, b_ref, o_ref):
    d = deg_ref[0][:, 0:1]
    dinv = lax.rsqrt(d + 1.0)
    p = dinv * (tmp_ref[0] + xs_ref[0])
    h = jnp.dot(p, w_ref[...], preferred_element_type=jnp.float32,
                 precision=lax.Precision.HIGHEST)
    h = jnp.maximum(h + b_ref[0], 0.0)
    hs = h * dinv
    o_ref[0, 0] = hs[:, :128]
    o_ref[0, 1] = hs[:, 128:]


_conv1 = pl.pallas_call(
    _conv1_body,
    grid=(2, _NB),
    in_specs=[pl.BlockSpec((1, _BN, IN_DIM), lambda g, i: (g, i, 0)),
              pl.BlockSpec((1, _BN, IN_DIM), lambda g, i: (g, i, 0)),
              pl.BlockSpec((1, _BN, 128), lambda g, i: (g, i, 0)),
              pl.BlockSpec((IN_DIM, HID), lambda g, i: (0, 0)),
              pl.BlockSpec((1, HID), lambda g, i: (0, 0))],
    out_specs=pl.BlockSpec((1, 2, _BN, 128), lambda g, i: (g, 0, i, 0)),
    out_shape=jax.ShapeDtypeStruct((2, 2, N, 128), jnp.float32),
)


def _conv2_pool_body(tmp_ref, hs_ref, deg_ref, w_ref, b_ref, batch_ref, o_ref):
    i = pl.program_id(1)
    d = deg_ref[0][:, 0:1]
    dinv = lax.rsqrt(d + 1.0)
    p_lo = dinv * (tmp_ref[0, 0] + hs_ref[0, 0])
    p_hi = dinv * (tmp_ref[0, 1] + hs_ref[0, 1])
    h2 = jnp.dot(p_lo, w_ref[:128, :], preferred_element_type=jnp.float32,
                 precision=lax.Precision.HIGHEST)
    h2 = h2 + jnp.dot(p_hi, w_ref[128:, :], preferred_element_type=jnp.float32,
                 precision=lax.Precision.HIGHEST)
    h2 = jnp.maximum(h2 + b_ref[0], 0.0)
    bvec = batch_ref[0, 0, 0, :]
    seg = lax.broadcasted_iota(jnp.int32, (B, _BN), 0)
    m = (seg == bvec[None, :]).astype(jnp.float32)
    h2e = jnp.concatenate([h2, jnp.ones((_BN, 128), jnp.float32)], axis=1)
    part = jnp.dot(m, h2e, preferred_element_type=jnp.float32,
                 precision=lax.Precision.HIGHEST)

    @pl.when(i == 0)
    def _():
        o_ref[0] = part

    @pl.when(i > 0)
    def _():
        o_ref[0] += part


_conv2_pool = pl.pallas_call(
    _conv2_pool_body,
    grid=(2, _NB),
    in_specs=[pl.BlockSpec((1, 2, _BN, 128), lambda g, i: (g, 0, i, 0)),
              pl.BlockSpec((1, 2, _BN, 128), lambda g, i: (g, 0, i, 0)),
              pl.BlockSpec((1, _BN, 128), lambda g, i: (g, i, 0)),
              pl.BlockSpec((HID, HID), lambda g, i: (0, 0)),
              pl.BlockSpec((1, HID), lambda g, i: (0, 0)),
              pl.BlockSpec((1, 1, 1, _BN), lambda g, i: (g, i, 0, 0))],
    out_specs=pl.BlockSpec((1, B, HID + 128), lambda g, i: (g, 0, 0)),
    out_shape=jax.ShapeDtypeStruct((2, B, HID + 128), jnp.float32),
)


def _cls_body(s_ref, wc1_ref, bc1_ref, wc2_ref, bc2_ref, o_ref):
    s1 = s_ref[0]
    s2 = s_ref[1]
    g1 = s1[:, :HID] / jnp.maximum(s1[:, HID:HID + 1], 1.0)
    g2 = s2[:, :HID] / jnp.maximum(s2[:, HID:HID + 1], 1.0)
    pair = jnp.concatenate([g1 * g2, jnp.abs(g1 - g2)], axis=1)
    hid = jnp.dot(pair, wc1_ref[...], preferred_element_type=jnp.float32,
                 precision=lax.Precision.HIGHEST)
    hid = jnp.maximum(hid + bc1_ref[0], 0.0)
    r = jnp.dot(hid, wc2_ref[...], preferred_element_type=jnp.float32,
                 precision=lax.Precision.HIGHEST)
    r = r + bc2_ref[0]
    o_ref[...] = jnp.broadcast_to(r, (B, 128))


_cls = pl.pallas_call(
    _cls_body,
    out_shape=jax.ShapeDtypeStruct((B, 128), jnp.float32),
)


def kernel(x1, edge_index1, batch1, x2, edge_index2, batch2,
           W1, b1, W2, b2, Wc1, bc1, Wc2, bc2):
    src1, dst1 = edge_index1[0], edge_index1[1]
    src2, dst2 = edge_index2[0], edge_index2[1]

    dst_cat = jnp.concatenate([dst1, dst2])
    src_cat = jnp.concatenate([src1, src2 + N])
    src4 = jnp.concatenate([src1, src1 + N, src2 + 2 * N, src2 + 3 * N])
    dst4 = jnp.concatenate([dst1, dst1, dst2, dst2])

    deg = _deg_kernel(dst_cat).reshape(2, N, 128)
    X = jnp.stack([x1, x2])
    XS = _prescale(X, deg)
    tmp1 = _prop2(XS.reshape(2 * N, 128), src_cat, dst_cat)
    HS4 = _conv1(tmp1.reshape(2, N, 128), XS, deg, W1, b1.reshape(1, HID))
    tmp2 = _prop4(HS4.reshape(4 * N, 128), src4, dst4)
    batch4 = jnp.stack([batch1, batch2]).reshape(2, _NB, 1, _BN)
    sums = _conv2_pool(tmp2.reshape(2, 2, N, 128), HS4, deg, W2,
                       b2.reshape(1, HID), batch4)
    out = _cls(sums, Wc1, bc1.reshape(1, HID), Wc2, bc2.reshape(1, 1))
    return out[:, 0]


# speedup vs baseline: 19.2629x; 1.0131x over previous
import functools

import jax
import jax.numpy as jnp
from jax import lax
from jax.experimental import pallas as pl
from jax.experimental.pallas import tpu as pltpu
from jax.experimental.pallas import tpu_sc as plsc

N = 10000
E = 320000
B = 512
IN_DIM = 128
HID = 256

NC = 2
NS = 16
CH = 80
EPT = E // NS
NCH = EPT // CH
RPT0 = 640
RPT_LAST = N - 15 * RPT0
NBLK = 5
IBE = EPT // NBLK
BCH = IBE // CH

_mesh = plsc.VectorSubcoreMesh(core_axis_name="c", subcore_axis_name="s",
                               num_cores=NC, num_subcores=NS)


@functools.partial(
    pl.kernel,
    out_type=jax.ShapeDtypeStruct((NC * N, 128), jnp.float32),
    mesh=_mesh,
    scratch_types=[
        pltpu.VMEM((EPT,), jnp.int32),
        pltpu.VMEM((CH, 128), jnp.float32),
        pltpu.VMEM((CH, 128), jnp.float32),
        pltpu.VMEM_SHARED((N, 128), jnp.float32),
    ],
)
def _deg_kernel(dst_hbm, deg_out, idx_v, ones_v, zer_v, acc):
    c = lax.axis_index("c")
    s = lax.axis_index("s")

    def fill_ones(i, _):
        for k in range(8):
            ones_v[i, pl.ds(k * 16, 16)] = jnp.ones((16,), jnp.float32)
            zer_v[i, pl.ds(k * 16, 16)] = jnp.zeros((16,), jnp.float32)
        return 0

    lax.fori_loop(0, CH, fill_ones, 0)

    @pl.when(s < 15)
    def _():
        for k in range(RPT0 // CH):
            pltpu.sync_copy(zer_v, acc.at[pl.ds(s * RPT0 + k * CH, CH)])

    @pl.when(s == 15)
    def _():
        for k in range(RPT_LAST // CH):
            pltpu.sync_copy(zer_v, acc.at[pl.ds(15 * RPT0 + k * CH, CH)])

    plsc.subcore_barrier()

    pltpu.sync_copy(dst_hbm.at[pl.ds(c * E + s * EPT, EPT)], idx_v)

    def body(j, _):
        pltpu.sync_copy(ones_v, acc.at[idx_v.at[pl.ds(j * CH, CH)]],
                        add=True)
        return 0

    lax.fori_loop(0, NCH, body, 0)
    plsc.subcore_barrier()

    @pl.when(s < 15)
    def _():
        pltpu.sync_copy(acc.at[pl.ds(s * RPT0, RPT0)],
                        deg_out.at[pl.ds(c * N + s * RPT0, RPT0)])

    @pl.when(s == 15)
    def _():
        pltpu.sync_copy(acc.at[pl.ds(15 * RPT0, RPT_LAST)],
                        deg_out.at[pl.ds(c * N + 15 * RPT0, RPT_LAST)])


def _make_prop(n_jobs):
    n_rounds = n_jobs // NC

    @functools.partial(
        pl.kernel,
        out_type=jax.ShapeDtypeStruct((n_jobs * N, 128), jnp.float32),
        mesh=_mesh,
        scratch_types=[
            pltpu.VMEM((IBE,), jnp.int32),
            pltpu.VMEM((IBE,), jnp.int32),
            pltpu.VMEM((CH, 128), jnp.float32),
            pltpu.VMEM((CH, 128), jnp.float32),
            pltpu.VMEM((CH, 128), jnp.float32),
            pltpu.VMEM_SHARED((N, 128), jnp.float32),
            pltpu.SemaphoreType.DMA,
            pltpu.SemaphoreType.DMA,
        ],
    )
    def prop(table_hbm, src_hbm, dst_hbm, out_hbm,
             src_v, dst_v, rows_a, rows_b, zer_v, acc, sem_a, sem_b):
        c = lax.axis_index("c")
        s = lax.axis_index("s")

        def fill_zero(i, _):
            for k in range(8):
                zer_v[i, pl.ds(k * 16, 16)] = jnp.zeros((16,), jnp.float32)
            return 0

        lax.fori_loop(0, CH, fill_zero, 0)

        def zero_acc():
            @pl.when(s < 15)
            def _():
                for k in range(RPT0 // CH):
                    pltpu.sync_copy(zer_v, acc.at[pl.ds(s * RPT0 + k * CH, CH)])

            @pl.when(s == 15)
            def _():
                for k in range(RPT_LAST // CH):
                    pltpu.sync_copy(zer_v, acc.at[pl.ds(15 * RPT0 + k * CH, CH)])

        zero_acc()
        plsc.subcore_barrier()

        for r in range(n_rounds):
            job = r * NC + c
            base = job * E + s * EPT

            def gather(j, buf, sem):
                return pltpu.async_copy(
                    table_hbm.at[src_v.at[pl.ds(j * CH, CH)]], buf, sem)

            def scat(j, buf):
                pltpu.sync_copy(buf, acc.at[dst_v.at[pl.ds(j * CH, CH)]],
                                add=True)

            def blk(bi, _):
                bbase = base + bi * IBE
                pltpu.sync_copy(src_hbm.at[pl.ds(bbase, IBE)], src_v)
                pltpu.sync_copy(dst_hbm.at[pl.ds(bbase, IBE)], dst_v)
                gather(0, rows_a, sem_a)

                def body(i, _):
                    ja = 2 * i
                    gather(ja + 1, rows_b, sem_b)
                    pltpu.make_async_copy(
                        table_hbm.at[src_v.at[pl.ds(ja * CH, CH)]],
                        rows_a, sem_a).wait()
                    scat(ja, rows_a)

                    @pl.when(i + 1 < BCH // 2)
                    def _():
                        gather(ja + 2, rows_a, sem_a)

                    pltpu.make_async_copy(
                        table_hbm.at[src_v.at[pl.ds((ja + 1) * CH, CH)]],
                        rows_b, sem_b).wait()
                    scat(ja + 1, rows_b)
                    return 0

                lax.fori_loop(0, BCH // 2, body, 0)
                return 0

            lax.fori_loop(0, NBLK, blk, 0)
            plsc.subcore_barrier()

            @pl.when(s < 15)
            def _():
                pltpu.sync_copy(acc.at[pl.ds(s * RPT0, RPT0)],
                                out_hbm.at[pl.ds(job * N + s * RPT0, RPT0)])

            @pl.when(s == 15)
            def _():
                pltpu.sync_copy(acc.at[pl.ds(15 * RPT0, RPT_LAST)],
                                out_hbm.at[pl.ds(job * N + 15 * RPT0, RPT_LAST)])

            if r + 1 < n_rounds:
                zero_acc()
                plsc.subcore_barrier()

    return prop


_prop2 = _make_prop(2)


_BN = 2000
_NB = N // _BN


def _prescale_body(x_ref, deg_ref, o_ref):
    d = deg_ref[0][:, 0:1]
    dinv = lax.rsqrt(d + 1.0)
    o_ref[0] = x_ref[0] * dinv


_prescale = pl.pallas_call(
    _prescale_body,
    grid=(2, _NB),
    in_specs=[pl.BlockSpec((1, _BN, IN_DIM), lambda g, i: (g, i, 0)),
              pl.BlockSpec((1, _BN, 128), lambda g, i: (g, i, 0))],
    out_specs=pl.BlockSpec((1, _BN, IN_DIM), lambda g, i: (g, i, 0)),
    out_shape=jax.ShapeDtypeStruct((2, N, IN_DIM), jnp.float32),
)


def _conv1_body(tmp_ref, xs_ref, deg_ref, w_ref, b_ref, o_ref):
    d = deg_ref[0][:, 0:1]
    dinv = lax.rsqrt(d + 1.0)
    p = dinv * (tmp_ref[0] + xs_ref[0])
    h = jnp.dot(p, w_ref[...], preferred_element_type=jnp.float32,
                 precision=lax.Precision.HIGHEST)
    h = jnp.maximum(h + b_ref[0], 0.0)
    hs = h * dinv
    o_ref[0, 0] = hs[:, :128]
    o_ref[0, 1] = hs[:, 128:]


_conv1 = pl.pallas_call(
    _conv1_body,
    grid=(2, _NB),
    in_specs=[pl.BlockSpec((1, _BN, IN_DIM), lambda g, i: (g, i, 0)),
              pl.BlockSpec((1, _BN, IN_DIM), lambda g, i: (g, i, 0)),
              pl.BlockSpec((1, _BN, 128), lambda g, i: (g, i, 0)),
              pl.BlockSpec((IN_DIM, HID), lambda g, i: (0, 0)),
              pl.BlockSpec((1, HID), lambda g, i: (0, 0))],
    out_specs=pl.BlockSpec((1, 2, _BN, 128), lambda g, i: (g, 0, i, 0)),
    out_shape=jax.ShapeDtypeStruct((2, 2, N, 128), jnp.float32),
)


def _conv2_pool_body(tmp_ref, hs_ref, deg_ref, w_ref, b_ref, batch_ref, o_ref):
    i = pl.program_id(0)
    d = deg_ref[...][:, 0:1]
    dinv = lax.rsqrt(d + 1.0)
    p_lo = dinv * (tmp_ref[0] + hs_ref[0])
    p_hi = dinv * (tmp_ref[1] + hs_ref[1])
    h2 = jnp.dot(p_lo, w_ref[:128, :], preferred_element_type=jnp.float32,
                 precision=lax.Precision.HIGHEST)
    h2 = h2 + jnp.dot(p_hi, w_ref[128:, :], preferred_element_type=jnp.float32,
                 precision=lax.Precision.HIGHEST)
    h2 = jnp.maximum(h2 + b_ref[0], 0.0)
    bvec = batch_ref[0, 0, :]
    seg = lax.broadcasted_iota(jnp.int32, (B, _BN), 0)
    m = (seg == bvec[None, :]).astype(jnp.float32)
    h2e = jnp.concatenate([h2, jnp.ones((_BN, 128), jnp.float32)], axis=1)
    part = jnp.dot(m, h2e, preferred_element_type=jnp.float32,
                 precision=lax.Precision.HIGHEST)

    @pl.when(i == 0)
    def _():
        o_ref[...] = part

    @pl.when(i > 0)
    def _():
        o_ref[...] += part


_conv2_pool = pl.pallas_call(
    _conv2_pool_body,
    grid=(_NB,),
    in_specs=[pl.BlockSpec((2, _BN, 128), lambda i: (0, i, 0)),
              pl.BlockSpec((2, _BN, 128), lambda i: (0, i, 0)),
              pl.BlockSpec((_BN, 128), lambda i: (i, 0)),
              pl.BlockSpec((HID, HID), lambda i: (0, 0)),
              pl.BlockSpec((1, HID), lambda i: (0, 0)),
              pl.BlockSpec((1, 1, _BN), lambda i: (i, 0, 0))],
    out_specs=pl.BlockSpec((B, HID + 128), lambda i: (0, 0)),
    out_shape=jax.ShapeDtypeStruct((B, HID + 128), jnp.float32),
)


def _cls_body(s1_ref, s2_ref, wc1_ref, bc1_ref, wc2_ref, bc2_ref, o_ref):
    s1 = s1_ref[...]
    s2 = s2_ref[...]
    g1 = s1[:, :HID] / jnp.maximum(s1[:, HID:HID + 1], 1.0)
    g2 = s2[:, :HID] / jnp.maximum(s2[:, HID:HID + 1], 1.0)
    pair = jnp.concatenate([g1 * g2, jnp.abs(g1 - g2)], axis=1)
    hid = jnp.dot(pair, wc1_ref[...], preferred_element_type=jnp.float32,
                 precision=lax.Precision.HIGHEST)
    hid = jnp.maximum(hid + bc1_ref[0], 0.0)
    r = jnp.dot(hid, wc2_ref[...], preferred_element_type=jnp.float32,
                 precision=lax.Precision.HIGHEST)
    r = r + bc2_ref[0]
    o_ref[...] = jnp.broadcast_to(r, (B, 128))


_cls = pl.pallas_call(
    _cls_body,
    out_shape=jax.ShapeDtypeStruct((B, 128), jnp.float32),
)


def kernel(x1, edge_index1, batch1, x2, edge_index2, batch2,
           W1, b1, W2, b2, Wc1, bc1, Wc2, bc2):
    src1, dst1 = edge_index1[0], edge_index1[1]
    src2, dst2 = edge_index2[0], edge_index2[1]

    dst_cat = jnp.concatenate([dst1, dst2])
    src_cat = jnp.concatenate([src1, src2 + N])

    deg = _deg_kernel(dst_cat).reshape(2, N, 128)
    X = jnp.stack([x1, x2])
    XS = _prescale(X, deg)
    tmp1 = _prop2(XS.reshape(2 * N, 128), src_cat, dst_cat)
    HS4 = _conv1(tmp1.reshape(2, N, 128), XS, deg, W1, b1.reshape(1, HID))

    b2r = b2.reshape(1, HID)
    tmp2_1 = _prop2(HS4[0].reshape(2 * N, 128),
                    jnp.concatenate([src1, src1 + N]),
                    jnp.concatenate([dst1, dst1]))
    tmp2_2 = _prop2(HS4[1].reshape(2 * N, 128),
                    jnp.concatenate([src2, src2 + N]),
                    jnp.concatenate([dst2, dst2]))
    sums1 = _conv2_pool(tmp2_1.reshape(2, N, 128), HS4[0], deg[0], W2, b2r,
                        batch1.reshape(_NB, 1, _BN))
    sums2 = _conv2_pool(tmp2_2.reshape(2, N, 128), HS4[1], deg[1], W2, b2r,
                        batch2.reshape(_NB, 1, _BN))
    out = _cls(sums1, sums2, Wc1, bc1.reshape(1, HID), Wc2, bc2.reshape(1, 1))
    return out[:, 0]


# speedup vs baseline: 19.9310x; 1.0347x over previous
import functools

import jax
import jax.numpy as jnp
from jax import lax
from jax.experimental import pallas as pl
from jax.experimental.pallas import tpu as pltpu
from jax.experimental.pallas import tpu_sc as plsc

N = 10000
E = 320000
B = 512
IN_DIM = 128
HID = 256

NC = 2
NS = 16
CH = 80
EPT = E // NS
NCH = EPT // CH
RPT0 = 640
RPT_LAST = N - 15 * RPT0
NBLK = 5
IBE = EPT // NBLK
BCH = IBE // CH

_mesh = plsc.VectorSubcoreMesh(core_axis_name="c", subcore_axis_name="s",
                               num_cores=NC, num_subcores=NS)


@functools.partial(
    pl.kernel,
    out_type=jax.ShapeDtypeStruct((NC * N, 128), jnp.float32),
    mesh=_mesh,
    scratch_types=[
        pltpu.VMEM((EPT,), jnp.int32),
        pltpu.VMEM((CH, 128), jnp.float32),
        pltpu.VMEM((CH, 128), jnp.float32),
        pltpu.VMEM_SHARED((N, 128), jnp.float32),
    ],
)
def _deg_kernel(dst1_hbm, dst2_hbm, deg_out, idx_v, ones_v, zer_v, acc):
    c = lax.axis_index("c")
    s = lax.axis_index("s")

    def fill_ones(i, _):
        for k in range(8):
            ones_v[i, pl.ds(k * 16, 16)] = jnp.ones((16,), jnp.float32)
            zer_v[i, pl.ds(k * 16, 16)] = jnp.zeros((16,), jnp.float32)
        return 0

    lax.fori_loop(0, CH, fill_ones, 0)

    @pl.when(s < 15)
    def _():
        for k in range(RPT0 // CH):
            pltpu.sync_copy(zer_v, acc.at[pl.ds(s * RPT0 + k * CH, CH)])

    @pl.when(s == 15)
    def _():
        for k in range(RPT_LAST // CH):
            pltpu.sync_copy(zer_v, acc.at[pl.ds(15 * RPT0 + k * CH, CH)])

    plsc.subcore_barrier()

    @pl.when(c == 0)
    def _():
        pltpu.sync_copy(dst1_hbm.at[pl.ds(s * EPT, EPT)], idx_v)

    @pl.when(c == 1)
    def _():
        pltpu.sync_copy(dst2_hbm.at[pl.ds(s * EPT, EPT)], idx_v)

    def body(j, _):
        pltpu.sync_copy(ones_v, acc.at[idx_v.at[pl.ds(j * CH, CH)]],
                        add=True)
        return 0

    lax.fori_loop(0, NCH, body, 0)
    plsc.subcore_barrier()

    @pl.when(s < 15)
    def _():
        pltpu.sync_copy(acc.at[pl.ds(s * RPT0, RPT0)],
                        deg_out.at[pl.ds(c * N + s * RPT0, RPT0)])

    @pl.when(s == 15)
    def _():
        pltpu.sync_copy(acc.at[pl.ds(15 * RPT0, RPT_LAST)],
                        deg_out.at[pl.ds(c * N + 15 * RPT0, RPT_LAST)])


_PROP_SCRATCH = [
    pltpu.VMEM((IBE,), jnp.int32),
    pltpu.VMEM((IBE,), jnp.int32),
    pltpu.VMEM((CH, 128), jnp.float32),
    pltpu.VMEM((CH, 128), jnp.float32),
    pltpu.VMEM((CH, 128), jnp.float32),
    pltpu.VMEM_SHARED((N, 128), jnp.float32),
    pltpu.SemaphoreType.DMA,
    pltpu.SemaphoreType.DMA,
]


def _fill_zero_rows(zer_v):
    def fill_zero(i, _):
        for k in range(8):
            zer_v[i, pl.ds(k * 16, 16)] = jnp.zeros((16,), jnp.float32)
        return 0

    lax.fori_loop(0, CH, fill_zero, 0)


def _zero_acc(s, zer_v, acc):
    @pl.when(s < 15)
    def _():
        for k in range(RPT0 // CH):
            pltpu.sync_copy(zer_v, acc.at[pl.ds(s * RPT0 + k * CH, CH)])

    @pl.when(s == 15)
    def _():
        for k in range(RPT_LAST // CH):
            pltpu.sync_copy(zer_v, acc.at[pl.ds(15 * RPT0 + k * CH, CH)])


def _add_off(src_v, off):
    offv = jnp.full((16,), off, jnp.int32)

    def body(k, _):
        src_v[pl.ds(k * 16, 16)] = src_v[pl.ds(k * 16, 16)] + offv
        return 0

    lax.fori_loop(0, IBE // 16, body, 0)


def _prop_pipeline(table_hbm, src_v, dst_v, acc, rows_a, rows_b, sem_a, sem_b):
    def gather(j, buf, sem):
        return pltpu.async_copy(
            table_hbm.at[src_v.at[pl.ds(j * CH, CH)]], buf, sem)

    def scat(j, buf):
        pltpu.sync_copy(buf, acc.at[dst_v.at[pl.ds(j * CH, CH)]], add=True)

    gather(0, rows_a, sem_a)

    def body(i, _):
        ja = 2 * i
        gather(ja + 1, rows_b, sem_b)
        pltpu.make_async_copy(
            table_hbm.at[src_v.at[pl.ds(ja * CH, CH)]], rows_a, sem_a).wait()
        scat(ja, rows_a)

        @pl.when(i + 1 < BCH // 2)
        def _():
            gather(ja + 2, rows_a, sem_a)

        pltpu.make_async_copy(
            table_hbm.at[src_v.at[pl.ds((ja + 1) * CH, CH)]],
            rows_b, sem_b).wait()
        scat(ja + 1, rows_b)
        return 0

    lax.fori_loop(0, BCH // 2, body, 0)


def _prop_out(c, s, acc, out_hbm):
    @pl.when(s < 15)
    def _():
        pltpu.sync_copy(acc.at[pl.ds(s * RPT0, RPT0)],
                        out_hbm.at[pl.ds(c * N + s * RPT0, RPT0)])

    @pl.when(s == 15)
    def _():
        pltpu.sync_copy(acc.at[pl.ds(15 * RPT0, RPT_LAST)],
                        out_hbm.at[pl.ds(c * N + 15 * RPT0, RPT_LAST)])


@functools.partial(
    pl.kernel,
    out_type=jax.ShapeDtypeStruct((2 * N, 128), jnp.float32),
    mesh=_mesh,
    scratch_types=_PROP_SCRATCH,
)
def _prop1(table_hbm, src1_hbm, dst1_hbm, src2_hbm, dst2_hbm, out_hbm,
           src_v, dst_v, rows_a, rows_b, zer_v, acc, sem_a, sem_b):
    c = lax.axis_index("c")
    s = lax.axis_index("s")
    _fill_zero_rows(zer_v)
    _zero_acc(s, zer_v, acc)
    plsc.subcore_barrier()

    def blk(bi, _):
        bbase = s * EPT + bi * IBE

        @pl.when(c == 0)
        def _():
            pltpu.sync_copy(src1_hbm.at[pl.ds(bbase, IBE)], src_v)
            pltpu.sync_copy(dst1_hbm.at[pl.ds(bbase, IBE)], dst_v)

        @pl.when(c == 1)
        def _():
            pltpu.sync_copy(src2_hbm.at[pl.ds(bbase, IBE)], src_v)
            pltpu.sync_copy(dst2_hbm.at[pl.ds(bbase, IBE)], dst_v)

        _add_off(src_v, c * N)
        _prop_pipeline(table_hbm, src_v, dst_v, acc, rows_a, rows_b,
                       sem_a, sem_b)
        return 0

    lax.fori_loop(0, NBLK, blk, 0)
    plsc.subcore_barrier()
    _prop_out(c, s, acc, out_hbm)


def _make_prop2(g):
    @functools.partial(
        pl.kernel,
        out_type=jax.ShapeDtypeStruct((2 * N, 128), jnp.float32),
        mesh=_mesh,
        scratch_types=_PROP_SCRATCH,
    )
    def prop(table_hbm, src_hbm, dst_hbm, out_hbm,
             src_v, dst_v, rows_a, rows_b, zer_v, acc, sem_a, sem_b):
        c = lax.axis_index("c")
        s = lax.axis_index("s")
        _fill_zero_rows(zer_v)
        _zero_acc(s, zer_v, acc)
        plsc.subcore_barrier()

        def blk(bi, _):
            bbase = s * EPT + bi * IBE
            pltpu.sync_copy(src_hbm.at[pl.ds(bbase, IBE)], src_v)
            pltpu.sync_copy(dst_hbm.at[pl.ds(bbase, IBE)], dst_v)
            _add_off(src_v, g * 2 * N + c * N)
            _prop_pipeline(table_hbm, src_v, dst_v, acc, rows_a, rows_b,
                           sem_a, sem_b)
            return 0

        lax.fori_loop(0, NBLK, blk, 0)
        plsc.subcore_barrier()
        _prop_out(c, s, acc, out_hbm)

    return prop


_prop2g = (_make_prop2(0), _make_prop2(1))


_BN = 2000
_NB = N // _BN


def _prescale_body(x_ref, deg_ref, o_ref):
    d = deg_ref[0][:, 0:1]
    dinv = lax.rsqrt(d + 1.0)
    o_ref[0] = x_ref[0] * dinv


_prescale = pl.pallas_call(
    _prescale_body,
    grid=(2, _NB),
    in_specs=[pl.BlockSpec((1, _BN, IN_DIM), lambda g, i: (g, i, 0)),
              pl.BlockSpec((1, _BN, 128), lambda g, i: (g, i, 0))],
    out_specs=pl.BlockSpec((1, _BN, IN_DIM), lambda g, i: (g, i, 0)),
    out_shape=jax.ShapeDtypeStruct((2, N, IN_DIM), jnp.float32),
)


def _conv1_body(tmp_ref, xs_ref, deg_ref, w_ref, b_ref, o_ref):
    d = deg_ref[0][:, 0:1]
    dinv = lax.rsqrt(d + 1.0)
    p = dinv * (tmp_ref[0] + xs_ref[0])
    h = jnp.dot(p, w_ref[...], preferred_element_type=jnp.float32,
                 precision=lax.Precision.HIGHEST)
    h = jnp.maximum(h + b_ref[0], 0.0)
    hs = h * dinv
    o_ref[0, 0] = hs[:, :128]
    o_ref[0, 1] = hs[:, 128:]


_conv1 = pl.pallas_call(
    _conv1_body,
    grid=(2, _NB),
    in_specs=[pl.BlockSpec((1, _BN, IN_DIM), lambda g, i: (g, i, 0)),
              pl.BlockSpec((1, _BN, IN_DIM), lambda g, i: (g, i, 0)),
              pl.BlockSpec((1, _BN, 128), lambda g, i: (g, i, 0)),
              pl.BlockSpec((IN_DIM, HID), lambda g, i: (0, 0)),
              pl.BlockSpec((1, HID), lambda g, i: (0, 0))],
    out_specs=pl.BlockSpec((1, 2, _BN, 128), lambda g, i: (g, 0, i, 0)),
    out_shape=jax.ShapeDtypeStruct((2, 2, N, 128), jnp.float32),
)


def _conv2_pool_body(tmp_ref, hs_ref, deg_ref, w_ref, b_ref, batch_ref, o_ref):
    i = pl.program_id(0)
    d = deg_ref[0][:, 0:1]
    dinv = lax.rsqrt(d + 1.0)
    p_lo = dinv * (tmp_ref[0] + hs_ref[0, 0])
    p_hi = dinv * (tmp_ref[1] + hs_ref[0, 1])
    h2 = jnp.dot(p_lo, w_ref[:128, :], preferred_element_type=jnp.float32,
                 precision=lax.Precision.HIGHEST)
    h2 = h2 + jnp.dot(p_hi, w_ref[128:, :], preferred_element_type=jnp.float32,
                 precision=lax.Precision.HIGHEST)
    h2 = jnp.maximum(h2 + b_ref[0], 0.0)
    bvec = batch_ref[0, 0, :]
    seg = lax.broadcasted_iota(jnp.int32, (B, _BN), 0)
    m = (seg == bvec[None, :]).astype(jnp.bfloat16)
    h2e = jnp.concatenate([h2, jnp.ones((_BN, 128), jnp.float32)], axis=1)
    a1 = h2e.astype(jnp.bfloat16)
    a2 = (h2e - a1.astype(jnp.float32)).astype(jnp.bfloat16)
    part = (jnp.dot(m, a1, preferred_element_type=jnp.float32)
            + jnp.dot(m, a2, preferred_element_type=jnp.float32))

    @pl.when(i == 0)
    def _():
        o_ref[...] = part

    @pl.when(i > 0)
    def _():
        o_ref[...] += part


def _make_conv2_pool(g):
    return pl.pallas_call(
        _conv2_pool_body,
        grid=(_NB,),
        in_specs=[pl.BlockSpec((2, _BN, 128), lambda i: (0, i, 0)),
                  pl.BlockSpec((1, 2, _BN, 128), lambda i: (g, 0, i, 0)),
                  pl.BlockSpec((1, _BN, 128), lambda i: (g, i, 0)),
                  pl.BlockSpec((HID, HID), lambda i: (0, 0)),
                  pl.BlockSpec((1, HID), lambda i: (0, 0)),
                  pl.BlockSpec((1, 1, _BN), lambda i: (i, 0, 0))],
        out_specs=pl.BlockSpec((B, HID + 128), lambda i: (0, 0)),
        out_shape=jax.ShapeDtypeStruct((B, HID + 128), jnp.float32),
    )


_conv2_poolg = (_make_conv2_pool(0), _make_conv2_pool(1))


def _cls_body(s1_ref, s2_ref, wc1_ref, bc1_ref, wc2_ref, bc2_ref, o_ref):
    s1 = s1_ref[...]
    s2 = s2_ref[...]
    g1 = s1[:, :HID] / jnp.maximum(s1[:, HID:HID + 1], 1.0)
    g2 = s2[:, :HID] / jnp.maximum(s2[:, HID:HID + 1], 1.0)
    pair = jnp.concatenate([g1 * g2, jnp.abs(g1 - g2)], axis=1)
    hid = jnp.dot(pair, wc1_ref[...], preferred_element_type=jnp.float32,
                 precision=lax.Precision.HIGHEST)
    hid = jnp.maximum(hid + bc1_ref[0], 0.0)
    r = jnp.dot(hid, wc2_ref[...], preferred_element_type=jnp.float32,
                 precision=lax.Precision.HIGHEST)
    r = r + bc2_ref[0]
    o_ref[...] = jnp.broadcast_to(r, (B, 128))


_cls = pl.pallas_call(
    _cls_body,
    out_shape=jax.ShapeDtypeStruct((B, 128), jnp.float32),
)


def kernel(x1, edge_index1, batch1, x2, edge_index2, batch2,
           W1, b1, W2, b2, Wc1, bc1, Wc2, bc2):
    src1, dst1 = edge_index1[0], edge_index1[1]
    src2, dst2 = edge_index2[0], edge_index2[1]

    deg = _deg_kernel(dst1, dst2).reshape(2, N, 128)
    X = jnp.stack([x1, x2])
    XS = _prescale(X, deg)
    tmp1 = _prop1(XS.reshape(2 * N, 128), src1, dst1, src2, dst2)
    HS4 = _conv1(tmp1.reshape(2, N, 128), XS, deg, W1, b1.reshape(1, HID))

    b2r = b2.reshape(1, HID)
    HS4f = HS4.reshape(4 * N, 128)
    tmp2_1 = _prop2g[0](HS4f, src1, dst1)
    tmp2_2 = _prop2g[1](HS4f, src2, dst2)
    sums1 = _conv2_poolg[0](tmp2_1.reshape(2, N, 128), HS4, deg, W2, b2r,
                            batch1.reshape(_NB, 1, _BN))
    sums2 = _conv2_poolg[1](tmp2_2.reshape(2, N, 128), HS4, deg, W2, b2r,
                            batch2.reshape(_NB, 1, _BN))
    out = _cls(sums1, sums2, Wc1, bc1.reshape(1, HID), Wc2, bc2.reshape(1, 1))
    return out[:, 0]
